# baseline (speedup 1.0000x reference)
import jax
import jax.numpy as jnp
from jax import lax
from jax.experimental import pallas as pl
from jax.experimental.pallas import tpu as pltpu
from jax.experimental.pallas import tpu_sc as plsc

_B, _N_DB, _N_CLASSES, _N_OBJ = 32, 8732, 36, 16
_IMG_H = 512.0
_DS = 4.0
_THRESHOLD = 0.5
_NEG_POS = 3
_ALPHA = 1.0

_ROWP = 8832
_NB = 256
_CH = _ROWP // 16


def _dot(a, b):
    return jax.lax.dot_general(a, b, (((1,), (0,)), ((), ())),
                               preferred_element_type=jnp.float32)


def _match_kernel(db_t_ref, b5_ref, cls_ref,
                  gx1_ref, gy1_ref, gx2_ref, gy2_ref, pos_ref, call_ref,
                  cn_ref, k_ref):
    dbt = db_t_ref[...]
    pcx = dbt[0:1, :]
    pcy = dbt[1:2, :]
    pw = dbt[2:3, :]
    ph = dbt[3:4, :]
    scale = _IMG_H / _DS
    dbx1 = (pcx - pw * 0.5) * scale
    dby1 = (pcy - ph * 0.5) * scale
    dbx2 = (pcx + pw * 0.5) * scale
    dby2 = (pcy + ph * 0.5) * scale

    b5 = b5_ref[0]
    bx1 = b5[:, 0:1]
    by1 = b5[:, 1:2]
    bx2 = b5[:, 2:3]
    by2 = b5[:, 3:4]

    iw = jnp.maximum(jnp.minimum(bx2, dbx2) - jnp.maximum(bx1, dbx1), 0.0)
    ih = jnp.maximum(jnp.minimum(by2, dby2) - jnp.maximum(by1, dby1), 0.0)
    inter = iw * ih
    area_a = (bx2 - bx1) * (by2 - by1)
    area_b = (dbx2 - dbx1) * (dby2 - dby1)
    union = area_a + area_b - inter
    iou = inter / jnp.maximum(union, 1e-10)

    oio = jax.lax.broadcasted_iota(jnp.int32, (_N_OBJ, _N_DB), 0)
    cio = jax.lax.broadcasted_iota(jnp.int32, (_N_OBJ, _N_DB), 1)

    ov = jnp.max(iou, axis=0, keepdims=True)
    obj_each = jnp.min(jnp.where(iou == ov, oio, _N_OBJ), axis=0,
                       keepdims=True)
    rmax = jnp.max(iou, axis=1, keepdims=True)
    db_for_obj = jnp.min(jnp.where(iou == rmax, cio, _N_DB), axis=1,
                         keepdims=True)

    fmatch = cio == db_for_obj
    j_sel = jnp.max(jnp.where(fmatch, oio, -1), axis=0, keepdims=True)
    forced = j_sel >= 0
    obj_each = jnp.where(forced, j_sel, obj_each)
    ov = jnp.where(forced, 1.0, ov)

    onehot_f = (obj_each == oio).astype(jnp.float32)
    g5 = _dot(b5.T, onehot_f)
    label_f = jnp.where(ov < _THRESHOLD, 0.0, g5[4:5, :])
    posf = (label_f != 0.0).astype(jnp.float32)

    cls_t = cls_ref[0].T
    e = jnp.exp(cls_t)
    kio = jax.lax.broadcasted_iota(jnp.int32, (_N_CLASSES, _N_DB), 0)
    masked = jnp.where(kio == label_f.astype(jnp.int32), cls_t, 0.0)
    ones_row = jnp.ones((1, _N_CLASSES), jnp.float32)
    sums = _dot(ones_row, e)
    picked = _dot(ones_row, masked)
    conf_all = jnp.log(sums) - picked

    gx1_ref[0] = g5[0:1, :]
    gy1_ref[0] = g5[1:2, :]
    gx2_ref[0] = g5[2:3, :]
    gy2_ref[0] = g5[3:4, :]
    pos_ref[0] = posf
    call_ref[0] = conf_all

    cn = jnp.where(posf != 0.0, 0.0, conf_all)
    cn_ref[0] = jnp.concatenate(
        [cn, jnp.zeros((1, _ROWP - _N_DB), jnp.float32)], axis=1)
    npos = jnp.sum(posf, keepdims=True).reshape(1, 1)
    k_ref[0] = jnp.broadcast_to(npos * float(_NEG_POS), (1, 128))


def _hist_select(hc_ref, hs_ref, k, lane):
    zeros16 = jnp.zeros((16,), jnp.float32)

    coarse = zeros16
    for cb in range(16):
        acc = zeros16
        for j in range(16):
            acc = acc + hc_ref[pl.ds((cb * 16 + j) * 16, 16)]
        coarse = jnp.where(lane == cb, jnp.sum(acc), coarse)
    suf = lax.rev(jnp.cumsum(lax.rev(coarse, (0,))), (0,))
    cb_star = jnp.max(plsc.all_reduce_population_count(suf >= k) - 1)

    finec = zeros16
    for j in range(16):
        tot = jnp.sum(hc_ref[pl.ds((cb_star * 16 + j) * 16, 16)])
        finec = jnp.where(lane == j, tot, finec)
    above_coarse = jnp.sum(jnp.where(lane > cb_star, coarse, 0.0))
    suff = lax.rev(jnp.cumsum(lax.rev(finec, (0,))), (0,)) + above_coarse
    j_star = jnp.max(plsc.all_reduce_population_count(suff >= k) - 1)
    b_star = cb_star * 16 + j_star

    s_at = jnp.sum(jnp.where(lane == j_star, suff, 0.0))
    f_at = jnp.sum(jnp.where(lane == j_star, finec, 0.0))
    cnt_above = s_at - f_at

    def sbody(r, acc):
        return acc + hs_ref[pl.ds(r * 16, 16)]
    sum_above = jnp.sum(lax.fori_loop(b_star + 1, _NB, sbody, zeros16))
    return b_star, cnt_above, sum_above


def _mine_kernel(cn_hbm, k_hbm, out_hbm,
                 row_v, kv_v, h1c, h1s, h2c, h2s, out_v):
    c = lax.axis_index("c")
    s = lax.axis_index("s")
    w = s * 2 + c
    pltpu.sync_copy(cn_hbm.at[w], row_v)
    pltpu.sync_copy(k_hbm.at[w], kv_v)
    k = jnp.max(kv_v[pl.ds(0, 16)])

    lane = lax.iota(jnp.int32, 16)
    zeros16 = jnp.zeros((16,), jnp.float32)
    ones16 = jnp.ones((16,), jnp.float32)

    def zbody(i, _):
        h1c[pl.ds(i * 16, 16)] = zeros16
        h1s[pl.ds(i * 16, 16)] = zeros16
        h2c[pl.ds(i * 16, 16)] = zeros16
        h2s[pl.ds(i * 16, 16)] = zeros16
        return 0
    lax.fori_loop(0, _NB, zbody, 0)

    def mbody(i, carry):
        m, t = carry
        v = row_v[pl.ds(i * 16, 16)]
        return jnp.maximum(m, v), t + v
    m_v, t_v = lax.fori_loop(0, _CH, mbody, (zeros16, zeros16))
    hi = jnp.maximum(jnp.max(m_v), jnp.float32(1e-30))
    row_sum = jnp.sum(t_v)
    r = lax.bitcast_convert_type(
        jnp.int32(0x7EF311C3) - lax.bitcast_convert_type(hi, jnp.int32),
        jnp.float32)
    r = r * (2.0 - hi * r)
    r = r * (2.0 - hi * r)
    r = r * (2.0 - hi * r)
    scale = jnp.float32(_NB) * r

    def bbody(i, _):
        v = row_v[pl.ds(i * 16, 16)]
        idx = jnp.clip((v * scale).astype(jnp.int32), 0, _NB - 1)
        addr = idx * 16 + lane
        plsc.addupdate_scatter(h1c, [addr], ones16)
        plsc.addupdate_scatter(h1s, [addr], v)
        return 0
    lax.fori_loop(0, _CH, bbody, 0)

    k_sel = jnp.minimum(k, jnp.float32(_ROWP))
    b1, cnt_above, sum_above = _hist_select(h1c, h1s, k_sel, lane)
    kp = k_sel - cnt_above
    lo2 = b1.astype(jnp.float32) * (hi * jnp.float32(1.0 / _NB))
    scale2 = scale * jnp.float32(_NB)

    def cbody(i, _):
        v = row_v[pl.ds(i * 16, 16)]
        idx = jnp.clip((v * scale).astype(jnp.int32), 0, _NB - 1)
        inb = idx == b1
        idx2 = jnp.clip(((v - lo2) * scale2).astype(jnp.int32), 0, _NB - 1)
        addr = idx2 * 16 + lane
        plsc.addupdate_scatter(h2c, [addr], ones16, mask=inb)
        plsc.addupdate_scatter(h2s, [addr], v, mask=inb)
        return 0
    lax.fori_loop(0, _CH, cbody, 0)

    b2, cnt2_above, sum2_above = _hist_select(h2c, h2s, kp, lane)
    rem = kp - cnt2_above
    mid = lo2 + (b2.astype(jnp.float32) + 0.5) * (hi * jnp.float32(1.0 / (_NB * _NB)))
    hard = sum_above + sum2_above + rem * mid
    hard = jnp.where(k >= jnp.float32(_ROWP), row_sum, hard)

    out_v[...] = jnp.full((16,), hard, jnp.float32)
    pltpu.sync_copy(out_v, out_hbm.at[w])


def _loss_kernel(gx1_ref, gy1_ref, gx2_ref, gy2_ref, pos_ref, call_ref,
                 locs_ref, db_t_ref, hard_ref, out_ref):
    dbt = db_t_ref[...]
    pcx = dbt[0:1, :]
    pcy = dbt[1:2, :]
    rpw = 1.0 / dbt[2:3, :]
    rph = 1.0 / dbt[3:4, :]

    posf = pos_ref[:, 0, :]
    gx1 = gx1_ref[:, 0, :]
    gy1 = gy1_ref[:, 0, :]
    gx2 = gx2_ref[:, 0, :]
    gy2 = gy2_ref[:, 0, :]
    inv = _DS / _IMG_H
    gcx = (gx1 + gx2) * (0.5 * inv)
    gcy = (gy1 + gy2) * (0.5 * inv)
    gw = (gx2 - gx1) * inv
    gh = (gy2 - gy1) * inv
    t_x = (gcx - pcx) * (10.0 * rpw)
    t_y = (gcy - pcy) * (10.0 * rph)
    t_w = jnp.log(jnp.maximum(gw, 1e-8) * rpw) * 5.0
    t_h = jnp.log(jnp.maximum(gh, 1e-8) * rph) * 5.0

    sl1_t = jnp.zeros((1, 1), jnp.float32)
    for c, t in enumerate((t_x, t_y, t_w, t_h)):
        d = locs_ref[c] - t
        ad = jnp.abs(d)
        s = jnp.where(ad < 1.0, 0.5 * d * d, ad - 0.5)
        sl1_t = sl1_t + jnp.sum(s * posf, keepdims=True).reshape(1, 1)

    conf_all = call_ref[:, 0, :]
    npos_col = jnp.sum(posf, axis=1, keepdims=True)
    cpos_t = jnp.sum(conf_all * posf, keepdims=True).reshape(1, 1)

    conf_hard = jnp.sum(hard_ref[:, 0:1], keepdims=True).reshape(1, 1)
    npos_t = jnp.sum(npos_col, keepdims=True).reshape(1, 1)
    loc_loss = sl1_t / jnp.maximum(npos_t * 4.0, 1.0)
    conf_loss = (conf_hard + cpos_t) / jnp.maximum(npos_t, 1.0)
    out_ref[...] = _ALPHA * loc_loss + conf_loss


def kernel(locs_pred, cls_pred, boxes, labels, default_boxes):
    lp_t = jnp.transpose(locs_pred, (2, 0, 1))
    db_t = jnp.transpose(default_boxes)
    b5 = jnp.concatenate(
        [boxes, labels.astype(jnp.float32)[..., None]], axis=2)

    row_shape = jax.ShapeDtypeStruct((_B, 1, _N_DB), jnp.float32)
    row_spec = pl.BlockSpec((1, 1, _N_DB), lambda i: (i, 0, 0))
    rows = pl.pallas_call(
        _match_kernel,
        grid=(_B,),
        in_specs=[
            pl.BlockSpec((4, _N_DB), lambda i: (0, 0)),
            pl.BlockSpec((1, _N_OBJ, 5), lambda i: (i, 0, 0)),
            pl.BlockSpec((1, _N_DB, _N_CLASSES), lambda i: (i, 0, 0)),
        ],
        out_specs=[row_spec] * 6 + [
            pl.BlockSpec((1, 1, _ROWP), lambda i: (i, 0, 0)),
            pl.BlockSpec((1, 1, 128), lambda i: (i, 0, 0)),
        ],
        out_shape=[row_shape] * 6 + [
            jax.ShapeDtypeStruct((_B, 1, _ROWP), jnp.float32),
            jax.ShapeDtypeStruct((_B, 1, 128), jnp.float32),
        ],
    )(db_t, b5, cls_pred)

    mine = pl.kernel(
        _mine_kernel,
        mesh=plsc.VectorSubcoreMesh(core_axis_name="c", subcore_axis_name="s"),
        compiler_params=pltpu.CompilerParams(needs_layout_passes=False),
        out_type=jax.ShapeDtypeStruct((_B, 16), jnp.float32),
        scratch_types=[
            pltpu.VMEM((_ROWP,), jnp.float32),
            pltpu.VMEM((128,), jnp.float32),
            pltpu.VMEM((_NB * 16,), jnp.float32),
            pltpu.VMEM((_NB * 16,), jnp.float32),
            pltpu.VMEM((_NB * 16,), jnp.float32),
            pltpu.VMEM((_NB * 16,), jnp.float32),
            pltpu.VMEM((16,), jnp.float32),
        ],
    )
    hard = mine(rows[6].reshape(_B, _ROWP), rows[7].reshape(_B, 128))

    loss = pl.pallas_call(
        _loss_kernel,
        out_shape=jax.ShapeDtypeStruct((1, 1), jnp.float32),
    )(*rows[:6], lp_t, db_t, hard)
    return loss[0, 0]


# speedup vs baseline: 1.7487x; 1.7487x over previous
import jax
import jax.numpy as jnp
from jax import lax
from jax.experimental import pallas as pl
from jax.experimental.pallas import tpu as pltpu
from jax.experimental.pallas import tpu_sc as plsc

_B, _N_DB, _N_CLASSES, _N_OBJ = 32, 8732, 36, 16
_IMG_H = 512.0
_DS = 4.0
_THRESHOLD = 0.5
_NEG_POS = 3
_ALPHA = 1.0

_ROWP = 8832
_NB = 256
_CH = _ROWP // 16


def _dot(a, b):
    return jax.lax.dot_general(a, b, (((1,), (0,)), ((), ())),
                               preferred_element_type=jnp.float32)


def _match_one(dbg, b5, cls_t, lp):
    (pcx, pcy, pw, ph, dbx1, dby1, dbx2, dby2) = dbg
    bx1 = b5[:, 0:1]
    by1 = b5[:, 1:2]
    bx2 = b5[:, 2:3]
    by2 = b5[:, 3:4]

    iw = jnp.maximum(jnp.minimum(bx2, dbx2) - jnp.maximum(bx1, dbx1), 0.0)
    ih = jnp.maximum(jnp.minimum(by2, dby2) - jnp.maximum(by1, dby1), 0.0)
    inter = iw * ih
    area_a = (bx2 - bx1) * (by2 - by1)
    area_b = (dbx2 - dbx1) * (dby2 - dby1)
    union = area_a + area_b - inter
    iou = inter / jnp.maximum(union, 1e-10)

    oio = jax.lax.broadcasted_iota(jnp.int32, (_N_OBJ, _N_DB), 0)
    cio = jax.lax.broadcasted_iota(jnp.int32, (_N_OBJ, _N_DB), 1)

    ov = jnp.max(iou, axis=0, keepdims=True)
    obj_each = jnp.min(jnp.where(iou == ov, oio, _N_OBJ), axis=0,
                       keepdims=True)
    rmax = jnp.max(iou, axis=1, keepdims=True)
    db_for_obj = jnp.min(jnp.where(iou == rmax, cio, _N_DB), axis=1,
                         keepdims=True)

    fmatch = cio == db_for_obj
    j_sel = jnp.max(jnp.where(fmatch, oio, -1), axis=0, keepdims=True)
    forced = j_sel >= 0
    obj_each = jnp.where(forced, j_sel, obj_each)
    ov = jnp.where(forced, 1.0, ov)

    onehot_f = (obj_each == oio).astype(jnp.float32)
    g5 = _dot(b5.T, onehot_f)
    label_f = jnp.where(ov < _THRESHOLD, 0.0, g5[4:5, :])
    posf = (label_f != 0.0).astype(jnp.float32)

    e = jnp.exp(cls_t)
    kio = jax.lax.broadcasted_iota(jnp.int32, (_N_CLASSES, _N_DB), 0)
    masked = jnp.where(kio == label_f.astype(jnp.int32), cls_t, 0.0)
    ones_row = jnp.ones((1, _N_CLASSES), jnp.float32)
    sums = _dot(ones_row, e)
    picked = _dot(ones_row, masked)
    conf_all = jnp.log(sums) - picked

    cn = jnp.where(posf != 0.0, 0.0, conf_all)
    cn_row = jnp.concatenate(
        [cn, jnp.zeros((1, _ROWP - _N_DB), jnp.float32)], axis=1)

    inv = _DS / _IMG_H
    gcx = (g5[0:1, :] + g5[2:3, :]) * (0.5 * inv)
    gcy = (g5[1:2, :] + g5[3:4, :]) * (0.5 * inv)
    gw = (g5[2:3, :] - g5[0:1, :]) * inv
    gh = (g5[3:4, :] - g5[1:2, :]) * inv
    rpw = 1.0 / pw
    rph = 1.0 / ph
    t_x = (gcx - pcx) * (10.0 * rpw)
    t_y = (gcy - pcy) * (10.0 * rph)
    t_w = jnp.log(jnp.maximum(gw, 1e-8) * rpw) * 5.0
    t_h = jnp.log(jnp.maximum(gh, 1e-8) * rph) * 5.0

    sl1 = jnp.zeros((1, 1), jnp.float32)
    for c, t in enumerate((t_x, t_y, t_w, t_h)):
        d = lp[c:c + 1, :] - t
        ad = jnp.abs(d)
        s = jnp.where(ad < 1.0, 0.5 * d * d, ad - 0.5)
        sl1 = sl1 + jnp.sum(s * posf, keepdims=True).reshape(1, 1)

    cpos = jnp.sum(conf_all * posf, keepdims=True).reshape(1, 1)
    npos = jnp.sum(posf, keepdims=True).reshape(1, 1)

    lane = jax.lax.broadcasted_iota(jnp.int32, (1, 128), 1)
    misc = jnp.where(lane < 16, npos * float(_NEG_POS), 0.0)
    misc = jnp.where(lane == 16, sl1, misc)
    misc = jnp.where(lane == 17, cpos, misc)
    misc = jnp.where(lane == 18, npos, misc)
    return cn_row, misc


_GPB = 8


def _match_kernel(db_t_ref, b5_ref, cls_ref, lp_ref, cn_ref, misc_ref):
    dbt = db_t_ref[...]
    pcx = dbt[0:1, :]
    pcy = dbt[1:2, :]
    pw = dbt[2:3, :]
    ph = dbt[3:4, :]
    scale = _IMG_H / _DS
    dbx1 = (pcx - pw * 0.5) * scale
    dby1 = (pcy - ph * 0.5) * scale
    dbx2 = (pcx + pw * 0.5) * scale
    dby2 = (pcy + ph * 0.5) * scale
    dbg = (pcx, pcy, pw, ph, dbx1, dby1, dbx2, dby2)
    for j in range(_GPB):
        cn_row, misc = _match_one(dbg, b5_ref[j], cls_ref[:, j, :], lp_ref[j])
        cn_ref[j] = cn_row
        misc_ref[j] = misc


def _hist_select(hc_ref, hs_ref, k, lane):
    zeros16 = jnp.zeros((16,), jnp.float32)

    coarse = zeros16
    for cb in range(16):
        acc = zeros16
        for j in range(16):
            acc = acc + hc_ref[pl.ds((cb * 16 + j) * 16, 16)]
        coarse = jnp.where(lane == cb, jnp.sum(acc), coarse)
    suf = lax.rev(jnp.cumsum(lax.rev(coarse, (0,))), (0,))
    cb_star = jnp.max(plsc.all_reduce_population_count(suf >= k) - 1)

    finec = zeros16
    for j in range(16):
        tot = jnp.sum(hc_ref[pl.ds((cb_star * 16 + j) * 16, 16)])
        finec = jnp.where(lane == j, tot, finec)
    above_coarse = jnp.sum(jnp.where(lane > cb_star, coarse, 0.0))
    suff = lax.rev(jnp.cumsum(lax.rev(finec, (0,))), (0,)) + above_coarse
    j_star = jnp.max(plsc.all_reduce_population_count(suff >= k) - 1)
    b_star = cb_star * 16 + j_star

    s_at = jnp.sum(jnp.where(lane == j_star, suff, 0.0))
    f_at = jnp.sum(jnp.where(lane == j_star, finec, 0.0))
    cnt_above = s_at - f_at

    def sbody(r, acc):
        return acc + hs_ref[pl.ds(r * 16, 16)]
    sum_above = jnp.sum(lax.fori_loop(b_star + 1, _NB, sbody, zeros16))
    return b_star, cnt_above, sum_above


def _mine_kernel(cn_hbm, k_hbm, out_hbm,
                 row_v, kv_v, h1c, h1s, h2c, h2s, out_v):
    c = lax.axis_index("c")
    s = lax.axis_index("s")
    w = s * 2 + c
    pltpu.sync_copy(cn_hbm.at[w], row_v)
    pltpu.sync_copy(k_hbm.at[w], kv_v)
    k = jnp.max(kv_v[pl.ds(0, 16)])

    lane = lax.iota(jnp.int32, 16)
    zeros16 = jnp.zeros((16,), jnp.float32)
    ones16 = jnp.ones((16,), jnp.float32)

    def zbody(i, _):
        h1c[pl.ds(i * 16, 16)] = zeros16
        h1s[pl.ds(i * 16, 16)] = zeros16
        h2c[pl.ds(i * 16, 16)] = zeros16
        h2s[pl.ds(i * 16, 16)] = zeros16
        return 0
    lax.fori_loop(0, _NB, zbody, 0)

    def mbody(i, carry):
        m, t = carry
        v = row_v[pl.ds(i * 16, 16)]
        return jnp.maximum(m, v), t + v
    m_v, t_v = lax.fori_loop(0, _CH, mbody, (zeros16, zeros16))
    hi = jnp.maximum(jnp.max(m_v), jnp.float32(1e-30))
    row_sum = jnp.sum(t_v)
    r = lax.bitcast_convert_type(
        jnp.int32(0x7EF311C3) - lax.bitcast_convert_type(hi, jnp.int32),
        jnp.float32)
    r = r * (2.0 - hi * r)
    r = r * (2.0 - hi * r)
    r = r * (2.0 - hi * r)
    scale = jnp.float32(_NB) * r

    def bbody(i, _):
        v = row_v[pl.ds(i * 16, 16)]
        idx = jnp.clip((v * scale).astype(jnp.int32), 0, _NB - 1)
        addr = idx * 16 + lane
        plsc.addupdate_scatter(h1c, [addr], ones16)
        plsc.addupdate_scatter(h1s, [addr], v)
        return 0
    lax.fori_loop(0, _CH, bbody, 0)

    k_sel = jnp.minimum(k, jnp.float32(_ROWP))
    b1, cnt_above, sum_above = _hist_select(h1c, h1s, k_sel, lane)
    kp = k_sel - cnt_above
    lo2 = b1.astype(jnp.float32) * (hi * jnp.float32(1.0 / _NB))
    scale2 = scale * jnp.float32(_NB)

    def cbody(i, _):
        v = row_v[pl.ds(i * 16, 16)]
        idx = jnp.clip((v * scale).astype(jnp.int32), 0, _NB - 1)
        inb = idx == b1
        idx2 = jnp.clip(((v - lo2) * scale2).astype(jnp.int32), 0, _NB - 1)
        addr = idx2 * 16 + lane
        plsc.addupdate_scatter(h2c, [addr], ones16, mask=inb)
        plsc.addupdate_scatter(h2s, [addr], v, mask=inb)
        return 0
    lax.fori_loop(0, _CH, cbody, 0)

    b2, cnt2_above, sum2_above = _hist_select(h2c, h2s, kp, lane)
    rem = kp - cnt2_above
    mid = lo2 + (b2.astype(jnp.float32) + 0.5) * (hi * jnp.float32(1.0 / (_NB * _NB)))
    hard = sum_above + sum2_above + rem * mid
    hard = jnp.where(k >= jnp.float32(_ROWP), row_sum, hard)

    out_v[...] = jnp.full((16,), hard, jnp.float32)
    pltpu.sync_copy(out_v, out_hbm.at[w])


def _combine_kernel(misc_ref, hard_ref, out_ref):
    m = misc_ref[:, 0, :]
    lane = jax.lax.broadcasted_iota(jnp.int32, (_B, 128), 1)
    sl1_t = jnp.sum(jnp.where(lane == 16, m, 0.0), keepdims=True).reshape(1, 1)
    cpos_t = jnp.sum(jnp.where(lane == 17, m, 0.0), keepdims=True).reshape(1, 1)
    npos_t = jnp.sum(jnp.where(lane == 18, m, 0.0), keepdims=True).reshape(1, 1)
    conf_hard = jnp.sum(hard_ref[:, 0:1], keepdims=True).reshape(1, 1)
    loc_loss = sl1_t / jnp.maximum(npos_t * 4.0, 1.0)
    conf_loss = (conf_hard + cpos_t) / jnp.maximum(npos_t, 1.0)
    out_ref[...] = _ALPHA * loc_loss + conf_loss


def kernel(locs_pred, cls_pred, boxes, labels, default_boxes):
    cls_t = jnp.transpose(cls_pred, (2, 0, 1))
    lp4 = jnp.transpose(locs_pred, (0, 2, 1))
    db_t = jnp.transpose(default_boxes)
    b5 = jnp.concatenate(
        [boxes, labels.astype(jnp.float32)[..., None]], axis=2)

    rows = pl.pallas_call(
        _match_kernel,
        grid=(_B // _GPB,),
        in_specs=[
            pl.BlockSpec((4, _N_DB), lambda i: (0, 0)),
            pl.BlockSpec((_GPB, _N_OBJ, 5), lambda i: (i, 0, 0)),
            pl.BlockSpec((_N_CLASSES, _GPB, _N_DB), lambda i: (0, i, 0)),
            pl.BlockSpec((_GPB, 4, _N_DB), lambda i: (i, 0, 0)),
        ],
        out_specs=[
            pl.BlockSpec((_GPB, 1, _ROWP), lambda i: (i, 0, 0)),
            pl.BlockSpec((_GPB, 1, 128), lambda i: (i, 0, 0)),
        ],
        out_shape=[
            jax.ShapeDtypeStruct((_B, 1, _ROWP), jnp.float32),
            jax.ShapeDtypeStruct((_B, 1, 128), jnp.float32),
        ],
    )(db_t, b5, cls_t, lp4)

    mine = pl.kernel(
        _mine_kernel,
        mesh=plsc.VectorSubcoreMesh(core_axis_name="c", subcore_axis_name="s"),
        compiler_params=pltpu.CompilerParams(needs_layout_passes=False),
        out_type=jax.ShapeDtypeStruct((_B, 16), jnp.float32),
        scratch_types=[
            pltpu.VMEM((_ROWP,), jnp.float32),
            pltpu.VMEM((128,), jnp.float32),
            pltpu.VMEM((_NB * 16,), jnp.float32),
            pltpu.VMEM((_NB * 16,), jnp.float32),
            pltpu.VMEM((_NB * 16,), jnp.float32),
            pltpu.VMEM((_NB * 16,), jnp.float32),
            pltpu.VMEM((16,), jnp.float32),
        ],
    )
    hard = mine(rows[0].reshape(_B, _ROWP), rows[1].reshape(_B, 128))

    loss = pl.pallas_call(
        _combine_kernel,
        out_shape=jax.ShapeDtypeStruct((1, 1), jnp.float32),
    )(rows[1], hard)
    return loss[0, 0]
